# baseline (speedup 1.0000x reference)
import jax
import jax.numpy as jnp
from jax import lax
from jax.experimental import pallas as pl
from jax.experimental.pallas import tpu as pltpu
from jax.experimental.pallas import tpu_sc as plsc

BATCH = 4
SEQ_LEN = 8192
D_MODEL = 768
BS = 2048
SEQ_TC = 6144
NC, NS, L = 2, 16, 16
NW = NC * NS
ROWS_PER_W = (SEQ_LEN - SEQ_TC) // NW
C = 32
NCHUNK = ROWS_PER_W // C
VECS_PER_ROW = D_MODEL // L


def _sc_body(in_hbm, pos_hbm, out_hbm,
             inb0, inb1, posb0, posb1,
             sin0, sin1, sout0, sout1, spos0, spos1):
    wid = lax.axis_index("s") * NC + lax.axis_index("c")
    base = SEQ_TC + wid * ROWS_PER_W
    inb = (inb0, inb1)
    posb = (posb0, posb1)
    sin = (sin0, sin1)
    sout = (sout0, sout1)
    spos = (spos0, spos1)

    h_in = [None, None]
    h_out = [None, None]
    h_pos = [None, None]

    NSTAGE = NCHUNK * BATCH

    h_pos[0] = pltpu.async_copy(pos_hbm.at[pl.ds(base, C)], posb[0], spos[0])
    h_in[0] = pltpu.async_copy(in_hbm.at[0, pl.ds(base, C)], inb[0], sin[0])

    for k in range(NSTAGE):
        ci, b, p = k // BATCH, k % BATCH, k % 2
        q = (k + 1) % 2
        if k + 1 < NSTAGE:
            ci1, b1 = (k + 1) // BATCH, (k + 1) % BATCH
            if h_out[q] is not None:
                h_out[q].wait()
                h_out[q] = None
            h_in[q] = pltpu.async_copy(
                in_hbm.at[b1, pl.ds(base + ci1 * C, C)], inb[q], sin[q])
        if b == 0 and ci + 1 < NCHUNK:
            pp = (ci + 1) % 2
            h_pos[pp] = pltpu.async_copy(
                pos_hbm.at[pl.ds(base + (ci + 1) * C, C)], posb[pp], spos[pp])
        h_in[p].wait()
        if b == 0:
            h_pos[ci % 2].wait()

        dst = inb[p]
        src = posb[ci % 2]

        def add_row(r, _):
            for j in range(VECS_PER_ROW):
                sl = pl.ds(j * L, L)
                dst[r, sl] = dst[r, sl] + src[r, sl]
            return ()

        lax.fori_loop(0, C, add_row, ())
        h_out[p] = pltpu.async_copy(
            dst, out_hbm.at[b, pl.ds(base + ci * C, C)], sout[p])

    h_out[0].wait()
    h_out[1].wait()


def _sc_part(inputs, pos_table):
    run = pl.kernel(
        _sc_body,
        out_type=jax.ShapeDtypeStruct((BATCH, SEQ_LEN, D_MODEL), jnp.float32),
        mesh=plsc.VectorSubcoreMesh(core_axis_name="c", subcore_axis_name="s"),
        scratch_types=[
            pltpu.VMEM((C, D_MODEL), jnp.float32),
            pltpu.VMEM((C, D_MODEL), jnp.float32),
            pltpu.VMEM((C, D_MODEL), jnp.float32),
            pltpu.VMEM((C, D_MODEL), jnp.float32),
            pltpu.SemaphoreType.DMA,
            pltpu.SemaphoreType.DMA,
            pltpu.SemaphoreType.DMA,
            pltpu.SemaphoreType.DMA,
            pltpu.SemaphoreType.DMA,
            pltpu.SemaphoreType.DMA,
        ],
    )
    return run(inputs, pos_table)


def _tc_add_kernel(x_ref, p_ref, a_ref, o_ref):
    del a_ref
    o_ref[...] = x_ref[...] + p_ref[...]


def _tc_part(inputs, pos_table, sc_out):
    grid = (SEQ_TC // BS, BATCH)
    return pl.pallas_call(
        _tc_add_kernel,
        grid=grid,
        in_specs=[
            pl.BlockSpec((1, BS, D_MODEL), lambda s, b: (b, s, 0)),
            pl.BlockSpec((BS, D_MODEL), lambda s, b: (s, 0)),
            pl.BlockSpec(memory_space=pl.ANY),
        ],
        out_specs=pl.BlockSpec((1, BS, D_MODEL), lambda s, b: (b, s, 0)),
        out_shape=jax.ShapeDtypeStruct((BATCH, SEQ_LEN, D_MODEL), jnp.float32),
        input_output_aliases={2: 0},
    )(inputs, pos_table, sc_out)


def kernel(inputs, pos_table):
    sc_out = _sc_part(inputs, pos_table)
    return _tc_part(inputs, pos_table, sc_out)
